# Initial kernel scaffold; baseline (speedup 1.0000x reference)
import jax
import jax.numpy as jnp
from jax.experimental import pallas as pl


def kernel(x, edge_index, W_init, W_f0, W_b0, W_sn0, W_f1, W_b1, W_sn1):
    raise NotImplementedError("write your pallas kernel here")


# speedup vs baseline: 2.5344x; 2.5344x over previous
import functools

import jax
import jax.numpy as jnp
from jax import lax
from jax.experimental import pallas as pl
from jax.experimental.pallas import tpu as pltpu
from jax.experimental.pallas import tpu_sc as plsc

N, E, D = 10000, 160000, 256
DC = 64
NCH = D // DC
NC, NS = 2, 16
ET = E // NS
B = 80
NB = ET // B
NP = 10240
DW = 16
ROWS_T = NP // NS
BN = 1000
GRID_N = N // BN


def _init_body(x_ref, w_ref, *o_refs):
    h = jax.nn.relu(jnp.dot(x_ref[...], w_ref[...],
                            preferred_element_type=jnp.float32))
    for k, o_ref in enumerate(o_refs):
        o_ref[...] = h[:, k * DC:(k + 1) * DC]


_init_mm = pl.pallas_call(
    _init_body,
    grid=(GRID_N,),
    in_specs=[pl.BlockSpec((BN, D), lambda i: (i, 0)),
              pl.BlockSpec((D, D), lambda i: (0, 0))],
    out_specs=[pl.BlockSpec((BN, DC), lambda i: (i, 0))] * NCH,
    out_shape=[jax.ShapeDtypeStruct((N, DC), jnp.float32)] * NCH,
)


def _wcomb_body(wf_ref, wb_ref, wm_ref, wt_ref, bm_ref, cm_ref):
    bm_ref[...] = jnp.dot(wf_ref[...], wm_ref[...],
                          preferred_element_type=jnp.float32)
    cm_ref[...] = jnp.dot(wb_ref[...], wt_ref[...],
                          preferred_element_type=jnp.float32)


_wcomb = pl.pallas_call(
    _wcomb_body,
    out_shape=[jax.ShapeDtypeStruct((D, D), jnp.float32),
               jax.ShapeDtypeStruct((D, D), jnp.float32)],
)


def _fuse_body(*refs):
    h_refs = refs[:NCH]
    s_refs = refs[NCH:2 * NCH]
    deg_ref, a_ref, b_ref, c_ref = refs[2 * NCH:2 * NCH + 4]
    out_refs = refs[2 * NCH + 4:]
    h_full = jnp.concatenate([r[...] for r in h_refs], axis=1)
    degf = jnp.maximum(deg_ref[:, 0:1], 1.0)
    degb = jnp.maximum(deg_ref[:, 1:2], 1.0)
    mf = jnp.concatenate([r[0] for r in s_refs], axis=1) / degf
    mb = jnp.concatenate([r[1] for r in s_refs], axis=1) / degb
    o = jnp.dot(h_full, a_ref[...], preferred_element_type=jnp.float32)
    o += jnp.dot(mf, b_ref[...], preferred_element_type=jnp.float32)
    o += jnp.dot(mb, c_ref[...], preferred_element_type=jnp.float32)
    o = jax.nn.relu(o)
    if len(out_refs) == NCH:
        for k, o_ref in enumerate(out_refs):
            o_ref[...] = o[:, k * DC:(k + 1) * DC]
    else:
        out_refs[0][...] = o


_fuse_in_specs = [
    pl.BlockSpec((BN, DC), lambda i: (i, 0)) for _ in range(NCH)
] + [
    pl.BlockSpec((NC, BN, DC), lambda i: (0, i, 0)) for _ in range(NCH)
] + [
    pl.BlockSpec((BN, NC), lambda i: (i, 0)),
    pl.BlockSpec((D, D), lambda i: (0, 0)),
    pl.BlockSpec((D, D), lambda i: (0, 0)),
    pl.BlockSpec((D, D), lambda i: (0, 0)),
]

_fuse_mid = pl.pallas_call(
    _fuse_body,
    grid=(GRID_N,),
    in_specs=_fuse_in_specs,
    out_specs=[pl.BlockSpec((BN, DC), lambda i: (i, 0))] * NCH,
    out_shape=[jax.ShapeDtypeStruct((N, DC), jnp.float32)] * NCH,
)

_fuse_last = pl.pallas_call(
    _fuse_body,
    grid=(GRID_N,),
    in_specs=_fuse_in_specs,
    out_specs=[pl.BlockSpec((BN, D), lambda i: (i, 0))],
    out_shape=[jax.ShapeDtypeStruct((N, D), jnp.float32)],
)


def _agg_body(*refs):
    h_chunks = refs[:NCH]
    idxg, idxs, zrows, zdeg, ones_h = refs[NCH:NCH + 5]
    sum_chunks = refs[NCH + 5:2 * NCH + 5]
    degout = refs[2 * NCH + 5]
    (idxg_v, idxs_v, rows0, rows1, ones_v, acc, dacc,
     sem0, sem1) = refs[2 * NCH + 6:]
    c = lax.axis_index("c")
    s = lax.axis_index("s")
    row0 = s * ROWS_T

    pltpu.sync_copy(idxg.at[c, s], idxg_v)
    pltpu.sync_copy(idxs.at[c, s], idxs_v)
    pltpu.sync_copy(ones_h, ones_v)

    for chunk in range(NCH):
        h_ref = h_chunks[chunk]
        out_ref = sum_chunks[chunk]

        pltpu.sync_copy(zrows.at[pl.ds(row0, ROWS_T)],
                        acc.at[pl.ds(row0, ROWS_T)])
        if chunk == 0:
            pltpu.sync_copy(zdeg.at[pl.ds(row0, ROWS_T)],
                            dacc.at[pl.ds(row0, ROWS_T)])
        plsc.subcore_barrier()

        pltpu.async_copy(h_ref.at[idxg_v.at[0]], rows0, sem0)

        def loop_body(b, carry, chunk=chunk, h_ref=h_ref):
            even = lax.rem(b, 2) == 0
            nxt = b + 1

            @pl.when(nxt < NB)
            def _():
                @pl.when(even)
                def _():
                    pltpu.async_copy(h_ref.at[idxg_v.at[nxt]], rows1, sem1)

                @pl.when(jnp.logical_not(even))
                def _():
                    pltpu.async_copy(h_ref.at[idxg_v.at[nxt]], rows0, sem0)

            @pl.when(even)
            def _():
                pltpu.make_async_copy(h_ref.at[idxg_v.at[b]], rows0,
                                      sem0).wait()
                pltpu.sync_copy(rows0, acc.at[idxs_v.at[b]], add=True)

            @pl.when(jnp.logical_not(even))
            def _():
                pltpu.make_async_copy(h_ref.at[idxg_v.at[b]], rows1,
                                      sem1).wait()
                pltpu.sync_copy(rows1, acc.at[idxs_v.at[b]], add=True)

            if chunk == 0:
                pltpu.sync_copy(ones_v, dacc.at[idxs_v.at[b]], add=True)
            return carry

        lax.fori_loop(0, NB, loop_body, 0)

        plsc.subcore_barrier()
        pltpu.sync_copy(acc.at[pl.ds(row0, ROWS_T)],
                        out_ref.at[c, pl.ds(row0, ROWS_T)])
        if chunk == 0:
            pltpu.sync_copy(dacc.at[pl.ds(row0, ROWS_T)],
                            degout.at[c, pl.ds(row0, ROWS_T)])
        plsc.subcore_barrier()


_agg = pl.kernel(
    _agg_body,
    out_type=[jax.ShapeDtypeStruct((NC, NP, DC), jnp.float32)] * NCH +
             [jax.ShapeDtypeStruct((NC, NP, DW), jnp.float32)],
    mesh=plsc.VectorSubcoreMesh(core_axis_name="c", subcore_axis_name="s",
                                num_cores=NC, num_subcores=NS),
    scratch_types=[pltpu.VMEM((NB, B), jnp.int32),
                   pltpu.VMEM((NB, B), jnp.int32),
                   pltpu.VMEM((B, DC), jnp.float32),
                   pltpu.VMEM((B, DC), jnp.float32),
                   pltpu.VMEM((B, DW), jnp.float32),
                   pltpu.VMEM_SHARED((NP, DC), jnp.float32),
                   pltpu.VMEM_SHARED((NP, DW), jnp.float32),
                   pltpu.SemaphoreType.DMA,
                   pltpu.SemaphoreType.DMA],
    compiler_params=pltpu.CompilerParams(use_tc_tiling_on_sc=False),
)


def kernel(x, edge_index, W_init, W_f0, W_b0, W_sn0, W_f1, W_b1, W_sn1):
    idxg = edge_index.reshape(NC, NS, NB, B)
    idxs = edge_index[::-1].reshape(NC, NS, NB, B)
    zrows = jnp.zeros((NP, DC), jnp.float32)
    zdeg = jnp.zeros((NP, DW), jnp.float32)
    ones_h = jnp.ones((B, DW), jnp.float32)

    hs = _init_mm(x, W_init)
    out = None
    for (Wf, Wb, Wsn, last) in ((W_f0, W_b0, W_sn0, False),
                                (W_f1, W_b1, W_sn1, True)):
        Bm, Cm = _wcomb(Wf, Wb, Wsn[D:2 * D], Wsn[2 * D:])
        aggout = _agg(*hs, idxg, idxs, zrows, zdeg, ones_h)
        ss, deg = aggout[:NCH], aggout[NCH]
        degT = deg[:, :, 0].T
        A = Wsn[:D]
        if last:
            (out,) = _fuse_last(*hs, *ss, degT, A, Bm, Cm)
        else:
            hs = _fuse_mid(*hs, *ss, degT, A, Bm, Cm)
    return out


# speedup vs baseline: 4.4788x; 1.7672x over previous
import functools

import jax
import jax.numpy as jnp
from jax import lax
from jax.experimental import pallas as pl
from jax.experimental.pallas import tpu as pltpu
from jax.experimental.pallas import tpu_sc as plsc

N, E, D = 10000, 160000, 256
DC = 64
NCH = D // DC
NC, NS = 2, 16
ET = E // NS
B = 80
NB = ET // B
NP = 10240
DW = 16
ROWS_T = NP // NS
BN = 1000
GRID_N = N // BN


def _init_body(x_ref, w_ref, *o_refs):
    h = jax.nn.relu(jnp.dot(x_ref[...], w_ref[...],
                            preferred_element_type=jnp.float32))
    for k, o_ref in enumerate(o_refs):
        o_ref[...] = h[:, k * DC:(k + 1) * DC]


_init_mm = pl.pallas_call(
    _init_body,
    grid=(GRID_N,),
    in_specs=[pl.BlockSpec((BN, D), lambda i: (i, 0)),
              pl.BlockSpec((D, D), lambda i: (0, 0))],
    out_specs=[pl.BlockSpec((BN, DC), lambda i: (i, 0))] * NCH,
    out_shape=[jax.ShapeDtypeStruct((N, DC), jnp.float32)] * NCH,
)


def _wcomb_body(wf_ref, wb_ref, wm_ref, wt_ref, bm_ref, cm_ref):
    bm_ref[...] = jnp.dot(wf_ref[...], wm_ref[...],
                          preferred_element_type=jnp.float32)
    cm_ref[...] = jnp.dot(wb_ref[...], wt_ref[...],
                          preferred_element_type=jnp.float32)


_wcomb = pl.pallas_call(
    _wcomb_body,
    out_shape=[jax.ShapeDtypeStruct((D, D), jnp.float32),
               jax.ShapeDtypeStruct((D, D), jnp.float32)],
)


def _fuse_body(*refs):
    h_refs = refs[:NCH]
    s_refs = refs[NCH:2 * NCH]
    deg_ref, a_ref, b_ref, c_ref = refs[2 * NCH:2 * NCH + 4]
    out_refs = refs[2 * NCH + 4:]
    h_full = jnp.concatenate([r[...] for r in h_refs], axis=1)
    degf = jnp.maximum(deg_ref[0, :, 0:1], 1.0)
    degb = jnp.maximum(deg_ref[1, :, 0:1], 1.0)
    mf = jnp.concatenate([r[0] for r in s_refs], axis=1) / degf
    mb = jnp.concatenate([r[1] for r in s_refs], axis=1) / degb
    o = jnp.dot(h_full, a_ref[...], preferred_element_type=jnp.float32)
    o += jnp.dot(mf, b_ref[...], preferred_element_type=jnp.float32)
    o += jnp.dot(mb, c_ref[...], preferred_element_type=jnp.float32)
    o = jax.nn.relu(o)
    if len(out_refs) == NCH:
        for k, o_ref in enumerate(out_refs):
            o_ref[...] = o[:, k * DC:(k + 1) * DC]
    else:
        out_refs[0][...] = o


_fuse_in_specs = [
    pl.BlockSpec((BN, DC), lambda i: (i, 0)) for _ in range(NCH)
] + [
    pl.BlockSpec((NC, BN, DC), lambda i: (0, i, 0)) for _ in range(NCH)
] + [
    pl.BlockSpec((NC, BN, DW), lambda i: (0, i, 0)),
    pl.BlockSpec((D, D), lambda i: (0, 0)),
    pl.BlockSpec((D, D), lambda i: (0, 0)),
    pl.BlockSpec((D, D), lambda i: (0, 0)),
]

_fuse_mid = pl.pallas_call(
    _fuse_body,
    grid=(GRID_N,),
    in_specs=_fuse_in_specs,
    out_specs=[pl.BlockSpec((BN, DC), lambda i: (i, 0))] * NCH,
    out_shape=[jax.ShapeDtypeStruct((N, DC), jnp.float32)] * NCH,
)

_fuse_last = pl.pallas_call(
    _fuse_body,
    grid=(GRID_N,),
    in_specs=_fuse_in_specs,
    out_specs=[pl.BlockSpec((BN, D), lambda i: (i, 0))],
    out_shape=[jax.ShapeDtypeStruct((N, D), jnp.float32)],
)


def _agg_body(*refs):
    h_chunks = refs[:NCH]
    idxg, zrows, zdeg, ones_h = refs[NCH:NCH + 4]
    sum_chunks = refs[NCH + 4:2 * NCH + 4]
    degout = refs[2 * NCH + 4]
    (idxg_v, idxs_v, rows0, rows1, ones_v, acc, dacc,
     sem0, sem1) = refs[2 * NCH + 5:]
    c = lax.axis_index("c")
    s = lax.axis_index("s")
    row0 = s * ROWS_T

    pltpu.sync_copy(idxg.at[c, s], idxg_v)
    pltpu.sync_copy(idxg.at[1 - c, s], idxs_v)
    pltpu.sync_copy(ones_h, ones_v)

    for chunk in range(NCH):
        h_ref = h_chunks[chunk]
        out_ref = sum_chunks[chunk]

        pltpu.sync_copy(zrows.at[pl.ds(row0, ROWS_T)],
                        acc.at[pl.ds(row0, ROWS_T)])
        if chunk == 0:
            pltpu.sync_copy(zdeg.at[pl.ds(row0, ROWS_T)],
                            dacc.at[pl.ds(row0, ROWS_T)])
        plsc.subcore_barrier()

        pltpu.async_copy(h_ref.at[idxg_v.at[0]], rows0, sem0)

        def loop_body(b, carry, chunk=chunk, h_ref=h_ref):
            even = lax.rem(b, 2) == 0
            nxt = b + 1

            @pl.when(nxt < NB)
            def _():
                @pl.when(even)
                def _():
                    pltpu.async_copy(h_ref.at[idxg_v.at[nxt]], rows1, sem1)

                @pl.when(jnp.logical_not(even))
                def _():
                    pltpu.async_copy(h_ref.at[idxg_v.at[nxt]], rows0, sem0)

            @pl.when(even)
            def _():
                pltpu.make_async_copy(h_ref.at[idxg_v.at[b]], rows0,
                                      sem0).wait()
                pltpu.sync_copy(rows0, acc.at[idxs_v.at[b]], add=True)

            @pl.when(jnp.logical_not(even))
            def _():
                pltpu.make_async_copy(h_ref.at[idxg_v.at[b]], rows1,
                                      sem1).wait()
                pltpu.sync_copy(rows1, acc.at[idxs_v.at[b]], add=True)

            if chunk == 0:
                pltpu.sync_copy(ones_v, dacc.at[idxs_v.at[b]], add=True)
            return carry

        lax.fori_loop(0, NB, loop_body, 0)

        plsc.subcore_barrier()
        pltpu.sync_copy(acc.at[pl.ds(row0, ROWS_T)],
                        out_ref.at[c, pl.ds(row0, ROWS_T)])
        if chunk == 0:
            pltpu.sync_copy(dacc.at[pl.ds(row0, ROWS_T)],
                            degout.at[c, pl.ds(row0, ROWS_T)])
        plsc.subcore_barrier()


_agg = pl.kernel(
    _agg_body,
    out_type=[jax.ShapeDtypeStruct((NC, NP, DC), jnp.float32)] * NCH +
             [jax.ShapeDtypeStruct((NC, NP, DW), jnp.float32)],
    mesh=plsc.VectorSubcoreMesh(core_axis_name="c", subcore_axis_name="s",
                                num_cores=NC, num_subcores=NS),
    scratch_types=[pltpu.VMEM((NB, B), jnp.int32),
                   pltpu.VMEM((NB, B), jnp.int32),
                   pltpu.VMEM((B, DC), jnp.float32),
                   pltpu.VMEM((B, DC), jnp.float32),
                   pltpu.VMEM((B, DW), jnp.float32),
                   pltpu.VMEM_SHARED((NP, DC), jnp.float32),
                   pltpu.VMEM_SHARED((NP, DW), jnp.float32),
                   pltpu.SemaphoreType.DMA,
                   pltpu.SemaphoreType.DMA],
    compiler_params=pltpu.CompilerParams(use_tc_tiling_on_sc=False),
)


def kernel(x, edge_index, W_init, W_f0, W_b0, W_sn0, W_f1, W_b1, W_sn1):
    idxg = edge_index.reshape(NC, NS, NB, B)
    zrows = jnp.zeros((NP, DC), jnp.float32)
    zdeg = jnp.zeros((NP, DW), jnp.float32)
    ones_h = jnp.ones((B, DW), jnp.float32)

    hs = _init_mm(x, W_init)
    out = None
    for (Wf, Wb, Wsn, last) in ((W_f0, W_b0, W_sn0, False),
                                (W_f1, W_b1, W_sn1, True)):
        Bm, Cm = _wcomb(Wf, Wb, Wsn[D:2 * D], Wsn[2 * D:])
        aggout = _agg(*hs, idxg, zrows, zdeg, ones_h)
        ss, deg = aggout[:NCH], aggout[NCH]
        A = Wsn[:D]
        if last:
            (out,) = _fuse_last(*hs, *ss, deg, A, Bm, Cm)
        else:
            hs = _fuse_mid(*hs, *ss, deg, A, Bm, Cm)
    return out


# speedup vs baseline: 6.0181x; 1.3437x over previous
import functools

import jax
import jax.numpy as jnp
from jax import lax
from jax.experimental import pallas as pl
from jax.experimental.pallas import tpu as pltpu
from jax.experimental.pallas import tpu_sc as plsc

N, E, D = 10000, 160000, 256
DC = 64
NCH = D // DC
NC, NS = 2, 16
ET = E // NS
B = 80
NB = ET // B
NP = 10240
DW = 16
ROWS_T = NP // NS
BN = 1000
GRID_N = N // BN


def _init_body(x_ref, w_ref, *o_refs):
    h = jax.nn.relu(jnp.dot(x_ref[...], w_ref[...],
                            preferred_element_type=jnp.float32))
    for k, o_ref in enumerate(o_refs):
        o_ref[...] = h[:, k * DC:(k + 1) * DC]


_init_mm = pl.pallas_call(
    _init_body,
    grid=(GRID_N,),
    in_specs=[pl.BlockSpec((BN, D), lambda i: (i, 0)),
              pl.BlockSpec((D, D), lambda i: (0, 0))],
    out_specs=[pl.BlockSpec((BN, DC), lambda i: (i, 0))] * NCH,
    out_shape=[jax.ShapeDtypeStruct((N, DC), jnp.float32)] * NCH,
)


def _wcomb_body(wf_ref, wb_ref, wm_ref, wt_ref, bm_ref, cm_ref):
    bm_ref[...] = jnp.dot(wf_ref[...], wm_ref[...],
                          preferred_element_type=jnp.float32)
    cm_ref[...] = jnp.dot(wb_ref[...], wt_ref[...],
                          preferred_element_type=jnp.float32)


_wcomb = pl.pallas_call(
    _wcomb_body,
    out_shape=[jax.ShapeDtypeStruct((D, D), jnp.float32),
               jax.ShapeDtypeStruct((D, D), jnp.float32)],
)


def _fuse_body(*refs):
    h_refs = refs[:NCH]
    s_refs = refs[NCH:2 * NCH]
    deg_ref, a_ref, b_ref, c_ref = refs[2 * NCH:2 * NCH + 4]
    out_refs = refs[2 * NCH + 4:]
    h_full = jnp.concatenate([r[...] for r in h_refs], axis=1)
    degf = jnp.maximum(deg_ref[0, :, 0:1], 1.0)
    degb = jnp.maximum(deg_ref[1, :, 0:1], 1.0)
    mf = jnp.concatenate([r[0] for r in s_refs], axis=1) / degf
    mb = jnp.concatenate([r[1] for r in s_refs], axis=1) / degb
    o = jnp.dot(h_full, a_ref[...], preferred_element_type=jnp.float32)
    o += jnp.dot(mf, b_ref[...], preferred_element_type=jnp.float32)
    o += jnp.dot(mb, c_ref[...], preferred_element_type=jnp.float32)
    o = jax.nn.relu(o)
    if len(out_refs) == NCH:
        for k, o_ref in enumerate(out_refs):
            o_ref[...] = o[:, k * DC:(k + 1) * DC]
    else:
        out_refs[0][...] = o


_fuse_in_specs = [
    pl.BlockSpec((BN, DC), lambda i: (i, 0)) for _ in range(NCH)
] + [
    pl.BlockSpec((NC, BN, DC), lambda i: (0, i, 0)) for _ in range(NCH)
] + [
    pl.BlockSpec((NC, BN, DW), lambda i: (0, i, 0)),
    pl.BlockSpec((D, D), lambda i: (0, 0)),
    pl.BlockSpec((D, D), lambda i: (0, 0)),
    pl.BlockSpec((D, D), lambda i: (0, 0)),
]

_fuse_mid = pl.pallas_call(
    _fuse_body,
    grid=(GRID_N,),
    in_specs=_fuse_in_specs,
    out_specs=[pl.BlockSpec((BN, DC), lambda i: (i, 0))] * NCH,
    out_shape=[jax.ShapeDtypeStruct((N, DC), jnp.float32)] * NCH,
)

_fuse_last = pl.pallas_call(
    _fuse_body,
    grid=(GRID_N,),
    in_specs=_fuse_in_specs,
    out_specs=[pl.BlockSpec((BN, D), lambda i: (i, 0))],
    out_shape=[jax.ShapeDtypeStruct((N, D), jnp.float32)],
)


NBUF = 4
PREF = 2


def _make_agg_body(want_deg):
    def _agg_body(*refs):
        h_chunks = refs[:NCH]
        if want_deg:
            idxg, zrows, zdeg, ones_h = refs[NCH:NCH + 4]
            sum_chunks = refs[NCH + 4:2 * NCH + 4]
            degout = refs[2 * NCH + 4]
            rest = refs[2 * NCH + 5:]
            idxg_v, idxs_v = rest[:2]
            rows = rest[2:2 + NBUF]
            ones_v, acc, dacc = rest[2 + NBUF:5 + NBUF]
            gsem = rest[5 + NBUF:5 + 2 * NBUF]
            ssem = rest[5 + 2 * NBUF:5 + 3 * NBUF]
        else:
            idxg, zrows = refs[NCH:NCH + 2]
            sum_chunks = refs[NCH + 2:2 * NCH + 2]
            rest = refs[2 * NCH + 2:]
            idxg_v, idxs_v = rest[:2]
            rows = rest[2:2 + NBUF]
            acc = rest[2 + NBUF]
            gsem = rest[3 + NBUF:3 + 2 * NBUF]
            ssem = rest[3 + 2 * NBUF:3 + 3 * NBUF]
        c = lax.axis_index("c")
        s = lax.axis_index("s")
        row0 = s * ROWS_T

        pltpu.sync_copy(idxg.at[c, s], idxg_v)
        pltpu.sync_copy(idxg.at[1 - c, s], idxs_v)
        if want_deg:
            pltpu.sync_copy(ones_h, ones_v)

        for chunk in range(NCH):
            h_ref = h_chunks[chunk]
            out_ref = sum_chunks[chunk]

            pltpu.sync_copy(zrows.at[pl.ds(row0, ROWS_T)],
                            acc.at[pl.ds(row0, ROWS_T)])
            if want_deg and chunk == 0:
                pltpu.sync_copy(zdeg.at[pl.ds(row0, ROWS_T)],
                                dacc.at[pl.ds(row0, ROWS_T)])
            plsc.subcore_barrier()

            for j in range(PREF):
                pltpu.async_copy(h_ref.at[idxg_v.at[j]], rows[j], gsem[j])

            def loop_body(b, carry, chunk=chunk, h_ref=h_ref):
                nxt = b + PREF

                @pl.when(nxt < NB)
                def _():
                    for j in range(NBUF):
                        @pl.when(lax.rem(nxt, NBUF) == j)
                        def _(j=j):
                            @pl.when(nxt >= NBUF)
                            def _():
                                pltpu.make_async_copy(
                                    rows[j], acc.at[idxs_v.at[b]],
                                    ssem[j]).wait()
                            pltpu.async_copy(h_ref.at[idxg_v.at[nxt]],
                                             rows[j], gsem[j])

                for j in range(NBUF):
                    @pl.when(lax.rem(b, NBUF) == j)
                    def _(j=j):
                        pltpu.make_async_copy(h_ref.at[idxg_v.at[b]],
                                              rows[j], gsem[j]).wait()
                        pltpu.async_copy(rows[j], acc.at[idxs_v.at[b]],
                                         ssem[j], add=True)

                if want_deg and chunk == 0:
                    pltpu.sync_copy(ones_v, dacc.at[idxs_v.at[b]], add=True)
                return carry

            lax.fori_loop(0, NB, loop_body, 0)

            for j in range(NBUF):
                pltpu.make_async_copy(rows[j], acc.at[idxs_v.at[0]],
                                      ssem[j]).wait()

            plsc.subcore_barrier()
            pltpu.sync_copy(acc.at[pl.ds(row0, ROWS_T)],
                            out_ref.at[c, pl.ds(row0, ROWS_T)])
            if want_deg and chunk == 0:
                pltpu.sync_copy(dacc.at[pl.ds(row0, ROWS_T)],
                                degout.at[c, pl.ds(row0, ROWS_T)])
            plsc.subcore_barrier()

    return _agg_body


_sc_mesh = plsc.VectorSubcoreMesh(core_axis_name="c", subcore_axis_name="s",
                                  num_cores=NC, num_subcores=NS)

_agg_deg = pl.kernel(
    _make_agg_body(True),
    out_type=[jax.ShapeDtypeStruct((NC, NP, DC), jnp.float32)] * NCH +
             [jax.ShapeDtypeStruct((NC, NP, DW), jnp.float32)],
    mesh=_sc_mesh,
    scratch_types=[pltpu.VMEM((NB, B), jnp.int32),
                   pltpu.VMEM((NB, B), jnp.int32)] +
                  [pltpu.VMEM((B, DC), jnp.float32)] * NBUF +
                  [pltpu.VMEM((B, DW), jnp.float32),
                   pltpu.VMEM_SHARED((NP, DC), jnp.float32),
                   pltpu.VMEM_SHARED((NP, DW), jnp.float32)] +
                  [pltpu.SemaphoreType.DMA] * (2 * NBUF),
    compiler_params=pltpu.CompilerParams(use_tc_tiling_on_sc=False),
)

_agg_nodeg = pl.kernel(
    _make_agg_body(False),
    out_type=[jax.ShapeDtypeStruct((NC, NP, DC), jnp.float32)] * NCH,
    mesh=_sc_mesh,
    scratch_types=[pltpu.VMEM((NB, B), jnp.int32),
                   pltpu.VMEM((NB, B), jnp.int32)] +
                  [pltpu.VMEM((B, DC), jnp.float32)] * NBUF +
                  [pltpu.VMEM_SHARED((NP, DC), jnp.float32)] +
                  [pltpu.SemaphoreType.DMA] * (2 * NBUF),
    compiler_params=pltpu.CompilerParams(use_tc_tiling_on_sc=False),
)


def kernel(x, edge_index, W_init, W_f0, W_b0, W_sn0, W_f1, W_b1, W_sn1):
    idxg = edge_index.reshape(NC, NS, NB, B)
    zrows = jnp.zeros((NP, DC), jnp.float32)
    zdeg = jnp.zeros((NP, DW), jnp.float32)
    ones_h = jnp.ones((B, DW), jnp.float32)

    hs = _init_mm(x, W_init)
    out = None
    deg = None
    for (Wf, Wb, Wsn, last) in ((W_f0, W_b0, W_sn0, False),
                                (W_f1, W_b1, W_sn1, True)):
        Bm, Cm = _wcomb(Wf, Wb, Wsn[D:2 * D], Wsn[2 * D:])
        if deg is None:
            aggout = _agg_deg(*hs, idxg, zrows, zdeg, ones_h)
            ss, deg = aggout[:NCH], aggout[NCH]
        else:
            ss = _agg_nodeg(*hs, idxg, zrows)
        A = Wsn[:D]
        if last:
            (out,) = _fuse_last(*hs, *ss, deg, A, Bm, Cm)
        else:
            hs = _fuse_mid(*hs, *ss, deg, A, Bm, Cm)
    return out


# speedup vs baseline: 6.1251x; 1.0178x over previous
import functools

import jax
import jax.numpy as jnp
from jax import lax
from jax.experimental import pallas as pl
from jax.experimental.pallas import tpu as pltpu
from jax.experimental.pallas import tpu_sc as plsc

N, E, D = 10000, 160000, 256
DC = 64
NCH = D // DC
NC, NS = 2, 16
ET = E // NS
B = 125
NB = ET // B
NP = 10240
DW = 16
ROWS_T = NP // NS
BN = 1000
GRID_N = N // BN


def _init_body(x_ref, w_ref, *o_refs):
    h = jax.nn.relu(jnp.dot(x_ref[...], w_ref[...],
                            preferred_element_type=jnp.float32))
    for k, o_ref in enumerate(o_refs):
        o_ref[...] = h[:, k * DC:(k + 1) * DC]


_init_mm = pl.pallas_call(
    _init_body,
    grid=(GRID_N,),
    in_specs=[pl.BlockSpec((BN, D), lambda i: (i, 0)),
              pl.BlockSpec((D, D), lambda i: (0, 0))],
    out_specs=[pl.BlockSpec((BN, DC), lambda i: (i, 0))] * NCH,
    out_shape=[jax.ShapeDtypeStruct((N, DC), jnp.float32)] * NCH,
)


def _wcomb_body(wf_ref, wb_ref, wm_ref, wt_ref, bm_ref, cm_ref):
    bm_ref[...] = jnp.dot(wf_ref[...], wm_ref[...],
                          preferred_element_type=jnp.float32)
    cm_ref[...] = jnp.dot(wb_ref[...], wt_ref[...],
                          preferred_element_type=jnp.float32)


_wcomb = pl.pallas_call(
    _wcomb_body,
    out_shape=[jax.ShapeDtypeStruct((D, D), jnp.float32),
               jax.ShapeDtypeStruct((D, D), jnp.float32)],
)


def _fuse_body(*refs):
    h_refs = refs[:NCH]
    s_refs = refs[NCH:2 * NCH]
    deg_ref, a_ref, b_ref, c_ref = refs[2 * NCH:2 * NCH + 4]
    out_refs = refs[2 * NCH + 4:]
    h_full = jnp.concatenate([r[...] for r in h_refs], axis=1)
    degf = jnp.maximum(deg_ref[0, :, 0:1], 1.0)
    degb = jnp.maximum(deg_ref[1, :, 0:1], 1.0)
    mf = jnp.concatenate([r[0] for r in s_refs], axis=1) / degf
    mb = jnp.concatenate([r[1] for r in s_refs], axis=1) / degb
    o = jnp.dot(h_full, a_ref[...], preferred_element_type=jnp.float32)
    o += jnp.dot(mf, b_ref[...], preferred_element_type=jnp.float32)
    o += jnp.dot(mb, c_ref[...], preferred_element_type=jnp.float32)
    o = jax.nn.relu(o)
    if len(out_refs) == NCH:
        for k, o_ref in enumerate(out_refs):
            o_ref[...] = o[:, k * DC:(k + 1) * DC]
    else:
        out_refs[0][...] = o


_fuse_in_specs = [
    pl.BlockSpec((BN, DC), lambda i: (i, 0)) for _ in range(NCH)
] + [
    pl.BlockSpec((NC, BN, DC), lambda i: (0, i, 0)) for _ in range(NCH)
] + [
    pl.BlockSpec((NC, BN, DW), lambda i: (0, i, 0)),
    pl.BlockSpec((D, D), lambda i: (0, 0)),
    pl.BlockSpec((D, D), lambda i: (0, 0)),
    pl.BlockSpec((D, D), lambda i: (0, 0)),
]

_fuse_mid = pl.pallas_call(
    _fuse_body,
    grid=(GRID_N,),
    in_specs=_fuse_in_specs,
    out_specs=[pl.BlockSpec((BN, DC), lambda i: (i, 0))] * NCH,
    out_shape=[jax.ShapeDtypeStruct((N, DC), jnp.float32)] * NCH,
)

_fuse_last = pl.pallas_call(
    _fuse_body,
    grid=(GRID_N,),
    in_specs=_fuse_in_specs,
    out_specs=[pl.BlockSpec((BN, D), lambda i: (i, 0))],
    out_shape=[jax.ShapeDtypeStruct((N, D), jnp.float32)],
)


NBUF = 4
PREF = 2


def _make_agg_body(want_deg):
    def _agg_body(*refs):
        h_chunks = refs[:NCH]
        if want_deg:
            idxg, zrows, zdeg, ones_h = refs[NCH:NCH + 4]
            sum_chunks = refs[NCH + 4:2 * NCH + 4]
            degout = refs[2 * NCH + 4]
            rest = refs[2 * NCH + 5:]
            idxg_v, idxs_v = rest[:2]
            rows = rest[2:2 + NBUF]
            ones_v, acc, dacc = rest[2 + NBUF:5 + NBUF]
            gsem = rest[5 + NBUF:5 + 2 * NBUF]
            ssem = rest[5 + 2 * NBUF:5 + 3 * NBUF]
        else:
            idxg, zrows = refs[NCH:NCH + 2]
            sum_chunks = refs[NCH + 2:2 * NCH + 2]
            rest = refs[2 * NCH + 2:]
            idxg_v, idxs_v = rest[:2]
            rows = rest[2:2 + NBUF]
            acc = rest[2 + NBUF]
            gsem = rest[3 + NBUF:3 + 2 * NBUF]
            ssem = rest[3 + 2 * NBUF:3 + 3 * NBUF]
        c = lax.axis_index("c")
        s = lax.axis_index("s")
        row0 = s * ROWS_T

        pltpu.sync_copy(idxg.at[c, s], idxg_v)
        pltpu.sync_copy(idxg.at[1 - c, s], idxs_v)
        if want_deg:
            pltpu.sync_copy(ones_h, ones_v)

        for chunk in range(NCH):
            h_ref = h_chunks[chunk]
            out_ref = sum_chunks[chunk]

            pltpu.sync_copy(zrows.at[pl.ds(row0, ROWS_T)],
                            acc.at[pl.ds(row0, ROWS_T)])
            if want_deg and chunk == 0:
                pltpu.sync_copy(zdeg.at[pl.ds(row0, ROWS_T)],
                                dacc.at[pl.ds(row0, ROWS_T)])
            plsc.subcore_barrier()

            for j in range(PREF):
                pltpu.async_copy(h_ref.at[idxg_v.at[j]], rows[j], gsem[j])

            def loop_body(b, carry, chunk=chunk, h_ref=h_ref):
                nxt = b + PREF

                @pl.when(nxt < NB)
                def _():
                    for j in range(NBUF):
                        @pl.when(lax.rem(nxt, NBUF) == j)
                        def _(j=j):
                            @pl.when(nxt >= NBUF)
                            def _():
                                pltpu.make_async_copy(
                                    rows[j], acc.at[idxs_v.at[b]],
                                    ssem[j]).wait()
                            pltpu.async_copy(h_ref.at[idxg_v.at[nxt]],
                                             rows[j], gsem[j])

                for j in range(NBUF):
                    @pl.when(lax.rem(b, NBUF) == j)
                    def _(j=j):
                        pltpu.make_async_copy(h_ref.at[idxg_v.at[b]],
                                              rows[j], gsem[j]).wait()
                        pltpu.async_copy(rows[j], acc.at[idxs_v.at[b]],
                                         ssem[j], add=True)

                if want_deg and chunk == 0:
                    pltpu.sync_copy(ones_v, dacc.at[idxs_v.at[b]], add=True)
                return carry

            lax.fori_loop(0, NB, loop_body, 0)

            for j in range(NBUF):
                pltpu.make_async_copy(rows[j], acc.at[idxs_v.at[0]],
                                      ssem[j]).wait()

            plsc.subcore_barrier()
            pltpu.sync_copy(acc.at[pl.ds(row0, ROWS_T)],
                            out_ref.at[c, pl.ds(row0, ROWS_T)])
            if want_deg and chunk == 0:
                pltpu.sync_copy(dacc.at[pl.ds(row0, ROWS_T)],
                                degout.at[c, pl.ds(row0, ROWS_T)])
            plsc.subcore_barrier()

    return _agg_body


_sc_mesh = plsc.VectorSubcoreMesh(core_axis_name="c", subcore_axis_name="s",
                                  num_cores=NC, num_subcores=NS)

_agg_deg = pl.kernel(
    _make_agg_body(True),
    out_type=[jax.ShapeDtypeStruct((NC, NP, DC), jnp.float32)] * NCH +
             [jax.ShapeDtypeStruct((NC, NP, DW), jnp.float32)],
    mesh=_sc_mesh,
    scratch_types=[pltpu.VMEM((NB, B), jnp.int32),
                   pltpu.VMEM((NB, B), jnp.int32)] +
                  [pltpu.VMEM((B, DC), jnp.float32)] * NBUF +
                  [pltpu.VMEM((B, DW), jnp.float32),
                   pltpu.VMEM_SHARED((NP, DC), jnp.float32),
                   pltpu.VMEM_SHARED((NP, DW), jnp.float32)] +
                  [pltpu.SemaphoreType.DMA] * (2 * NBUF),
    compiler_params=pltpu.CompilerParams(use_tc_tiling_on_sc=False),
)

_agg_nodeg = pl.kernel(
    _make_agg_body(False),
    out_type=[jax.ShapeDtypeStruct((NC, NP, DC), jnp.float32)] * NCH,
    mesh=_sc_mesh,
    scratch_types=[pltpu.VMEM((NB, B), jnp.int32),
                   pltpu.VMEM((NB, B), jnp.int32)] +
                  [pltpu.VMEM((B, DC), jnp.float32)] * NBUF +
                  [pltpu.VMEM_SHARED((NP, DC), jnp.float32)] +
                  [pltpu.SemaphoreType.DMA] * (2 * NBUF),
    compiler_params=pltpu.CompilerParams(use_tc_tiling_on_sc=False),
)


def kernel(x, edge_index, W_init, W_f0, W_b0, W_sn0, W_f1, W_b1, W_sn1):
    idxg = edge_index.reshape(NC, NS, NB, B)
    zrows = jnp.zeros((NP, DC), jnp.float32)
    zdeg = jnp.zeros((NP, DW), jnp.float32)
    ones_h = jnp.ones((B, DW), jnp.float32)

    hs = _init_mm(x, W_init)
    out = None
    deg = None
    for (Wf, Wb, Wsn, last) in ((W_f0, W_b0, W_sn0, False),
                                (W_f1, W_b1, W_sn1, True)):
        Bm, Cm = _wcomb(Wf, Wb, Wsn[D:2 * D], Wsn[2 * D:])
        if deg is None:
            aggout = _agg_deg(*hs, idxg, zrows, zdeg, ones_h)
            ss, deg = aggout[:NCH], aggout[NCH]
        else:
            ss = _agg_nodeg(*hs, idxg, zrows)
        A = Wsn[:D]
        if last:
            (out,) = _fuse_last(*hs, *ss, deg, A, Bm, Cm)
        else:
            hs = _fuse_mid(*hs, *ss, deg, A, Bm, Cm)
    return out


# speedup vs baseline: 6.3716x; 1.0403x over previous
import functools

import jax
import jax.numpy as jnp
from jax import lax
from jax.experimental import pallas as pl
from jax.experimental.pallas import tpu as pltpu
from jax.experimental.pallas import tpu_sc as plsc

N, E, D = 10000, 160000, 256
DC = 64
NCH = D // DC
NC, NS = 2, 16
ET = E // NS
B = 125
NB = ET // B
NP = 10240
DW = 16
ROWS_T = NP // NS
BN = 1000
GRID_N = N // BN


def _init_body(x_ref, w_ref, *o_refs):
    h = jax.nn.relu(jnp.dot(x_ref[...], w_ref[...],
                            preferred_element_type=jnp.float32))
    for k, o_ref in enumerate(o_refs):
        o_ref[...] = h[:, k * DC:(k + 1) * DC]


_init_mm = pl.pallas_call(
    _init_body,
    grid=(GRID_N,),
    in_specs=[pl.BlockSpec((BN, D), lambda i: (i, 0)),
              pl.BlockSpec((D, D), lambda i: (0, 0))],
    out_specs=[pl.BlockSpec((BN, DC), lambda i: (i, 0))] * NCH,
    out_shape=[jax.ShapeDtypeStruct((N, DC), jnp.float32)] * NCH,
)


def _wcomb_body(wf_ref, wb_ref, wm_ref, wt_ref, bm_ref, cm_ref):
    bm_ref[...] = jnp.dot(wf_ref[...], wm_ref[...],
                          preferred_element_type=jnp.float32)
    cm_ref[...] = jnp.dot(wb_ref[...], wt_ref[...],
                          preferred_element_type=jnp.float32)


_wcomb = pl.pallas_call(
    _wcomb_body,
    out_shape=[jax.ShapeDtypeStruct((D, D), jnp.float32),
               jax.ShapeDtypeStruct((D, D), jnp.float32)],
)


def _fuse_body(*refs):
    h_refs = refs[:NCH]
    s_refs = refs[NCH:2 * NCH]
    deg_ref, a_ref, b_ref, c_ref = refs[2 * NCH:2 * NCH + 4]
    out_refs = refs[2 * NCH + 4:]
    h_full = jnp.concatenate([r[...] for r in h_refs], axis=1)
    degf = jnp.maximum(deg_ref[0, :, 0:1], 1.0)
    degb = jnp.maximum(deg_ref[1, :, 0:1], 1.0)
    mf = jnp.concatenate([r[0] for r in s_refs], axis=1) / degf
    mb = jnp.concatenate([r[1] for r in s_refs], axis=1) / degb
    o = jnp.dot(h_full, a_ref[...], preferred_element_type=jnp.float32)
    o += jnp.dot(mf, b_ref[...], preferred_element_type=jnp.float32)
    o += jnp.dot(mb, c_ref[...], preferred_element_type=jnp.float32)
    o = jax.nn.relu(o)
    if len(out_refs) == NCH:
        for k, o_ref in enumerate(out_refs):
            o_ref[...] = o[:, k * DC:(k + 1) * DC]
    else:
        out_refs[0][...] = o


_fuse_in_specs = [
    pl.BlockSpec((BN, DC), lambda i: (i, 0)) for _ in range(NCH)
] + [
    pl.BlockSpec((NC, BN, DC), lambda i: (0, i, 0)) for _ in range(NCH)
] + [
    pl.BlockSpec((NC, BN, DW), lambda i: (0, i, 0)),
    pl.BlockSpec((D, D), lambda i: (0, 0)),
    pl.BlockSpec((D, D), lambda i: (0, 0)),
    pl.BlockSpec((D, D), lambda i: (0, 0)),
]

_fuse_mid = pl.pallas_call(
    _fuse_body,
    grid=(GRID_N,),
    in_specs=_fuse_in_specs,
    out_specs=[pl.BlockSpec((BN, DC), lambda i: (i, 0))] * NCH,
    out_shape=[jax.ShapeDtypeStruct((N, DC), jnp.float32)] * NCH,
)

_fuse_last = pl.pallas_call(
    _fuse_body,
    grid=(GRID_N,),
    in_specs=_fuse_in_specs,
    out_specs=[pl.BlockSpec((BN, D), lambda i: (i, 0))],
    out_shape=[jax.ShapeDtypeStruct((N, D), jnp.float32)],
)


NBUF = 6
PREF = 4


def _make_agg_body(want_deg):
    def _agg_body(*refs):
        h_chunks = refs[:NCH]
        if want_deg:
            idxg, zrows, zdeg, ones_h = refs[NCH:NCH + 4]
            sum_chunks = refs[NCH + 4:2 * NCH + 4]
            degout = refs[2 * NCH + 4]
            rest = refs[2 * NCH + 5:]
            idxg_v, idxs_v = rest[:2]
            rows = rest[2:2 + NBUF]
            ones_v, acc, dacc = rest[2 + NBUF:5 + NBUF]
            gsem = rest[5 + NBUF:5 + 2 * NBUF]
            ssem = rest[5 + 2 * NBUF:5 + 3 * NBUF]
        else:
            idxg, zrows = refs[NCH:NCH + 2]
            sum_chunks = refs[NCH + 2:2 * NCH + 2]
            rest = refs[2 * NCH + 2:]
            idxg_v, idxs_v = rest[:2]
            rows = rest[2:2 + NBUF]
            acc = rest[2 + NBUF]
            gsem = rest[3 + NBUF:3 + 2 * NBUF]
            ssem = rest[3 + 2 * NBUF:3 + 3 * NBUF]
        c = lax.axis_index("c")
        s = lax.axis_index("s")
        row0 = s * ROWS_T

        pltpu.sync_copy(idxg.at[c, s], idxg_v)
        pltpu.sync_copy(idxg.at[1 - c, s], idxs_v)
        if want_deg:
            pltpu.sync_copy(ones_h, ones_v)

        for chunk in range(NCH):
            h_ref = h_chunks[chunk]
            out_ref = sum_chunks[chunk]

            pltpu.sync_copy(zrows.at[pl.ds(row0, ROWS_T)],
                            acc.at[pl.ds(row0, ROWS_T)])
            if want_deg and chunk == 0:
                pltpu.sync_copy(zdeg.at[pl.ds(row0, ROWS_T)],
                                dacc.at[pl.ds(row0, ROWS_T)])
            plsc.subcore_barrier()

            for j in range(PREF):
                pltpu.async_copy(h_ref.at[idxg_v.at[j]], rows[j], gsem[j])

            def loop_body(b, carry, chunk=chunk, h_ref=h_ref):
                nxt = b + PREF

                @pl.when(nxt < NB)
                def _():
                    for j in range(NBUF):
                        @pl.when(lax.rem(nxt, NBUF) == j)
                        def _(j=j):
                            @pl.when(nxt >= NBUF)
                            def _():
                                pltpu.make_async_copy(
                                    rows[j], acc.at[idxs_v.at[b]],
                                    ssem[j]).wait()
                            pltpu.async_copy(h_ref.at[idxg_v.at[nxt]],
                                             rows[j], gsem[j])

                for j in range(NBUF):
                    @pl.when(lax.rem(b, NBUF) == j)
                    def _(j=j):
                        pltpu.make_async_copy(h_ref.at[idxg_v.at[b]],
                                              rows[j], gsem[j]).wait()
                        pltpu.async_copy(rows[j], acc.at[idxs_v.at[b]],
                                         ssem[j], add=True)

                if want_deg and chunk == 0:
                    pltpu.sync_copy(ones_v, dacc.at[idxs_v.at[b]], add=True)
                return carry

            lax.fori_loop(0, NB, loop_body, 0)

            for j in range(NBUF):
                pltpu.make_async_copy(rows[j], acc.at[idxs_v.at[0]],
                                      ssem[j]).wait()

            plsc.subcore_barrier()
            pltpu.sync_copy(acc.at[pl.ds(row0, ROWS_T)],
                            out_ref.at[c, pl.ds(row0, ROWS_T)])
            if want_deg and chunk == 0:
                pltpu.sync_copy(dacc.at[pl.ds(row0, ROWS_T)],
                                degout.at[c, pl.ds(row0, ROWS_T)])
            plsc.subcore_barrier()

    return _agg_body


_sc_mesh = plsc.VectorSubcoreMesh(core_axis_name="c", subcore_axis_name="s",
                                  num_cores=NC, num_subcores=NS)

_agg_deg = pl.kernel(
    _make_agg_body(True),
    out_type=[jax.ShapeDtypeStruct((NC, NP, DC), jnp.float32)] * NCH +
             [jax.ShapeDtypeStruct((NC, NP, DW), jnp.float32)],
    mesh=_sc_mesh,
    scratch_types=[pltpu.VMEM((NB, B), jnp.int32),
                   pltpu.VMEM((NB, B), jnp.int32)] +
                  [pltpu.VMEM((B, DC), jnp.float32)] * NBUF +
                  [pltpu.VMEM((B, DW), jnp.float32),
                   pltpu.VMEM_SHARED((NP, DC), jnp.float32),
                   pltpu.VMEM_SHARED((NP, DW), jnp.float32)] +
                  [pltpu.SemaphoreType.DMA] * (2 * NBUF),
    compiler_params=pltpu.CompilerParams(use_tc_tiling_on_sc=False),
)

_agg_nodeg = pl.kernel(
    _make_agg_body(False),
    out_type=[jax.ShapeDtypeStruct((NC, NP, DC), jnp.float32)] * NCH,
    mesh=_sc_mesh,
    scratch_types=[pltpu.VMEM((NB, B), jnp.int32),
                   pltpu.VMEM((NB, B), jnp.int32)] +
                  [pltpu.VMEM((B, DC), jnp.float32)] * NBUF +
                  [pltpu.VMEM_SHARED((NP, DC), jnp.float32)] +
                  [pltpu.SemaphoreType.DMA] * (2 * NBUF),
    compiler_params=pltpu.CompilerParams(use_tc_tiling_on_sc=False),
)


def kernel(x, edge_index, W_init, W_f0, W_b0, W_sn0, W_f1, W_b1, W_sn1):
    idxg = edge_index.reshape(NC, NS, NB, B)
    zrows = jnp.zeros((NP, DC), jnp.float32)
    zdeg = jnp.zeros((NP, DW), jnp.float32)
    ones_h = jnp.ones((B, DW), jnp.float32)

    hs = _init_mm(x, W_init)
    out = None
    deg = None
    for (Wf, Wb, Wsn, last) in ((W_f0, W_b0, W_sn0, False),
                                (W_f1, W_b1, W_sn1, True)):
        Bm, Cm = _wcomb(Wf, Wb, Wsn[D:2 * D], Wsn[2 * D:])
        if deg is None:
            aggout = _agg_deg(*hs, idxg, zrows, zdeg, ones_h)
            ss, deg = aggout[:NCH], aggout[NCH]
        else:
            ss = _agg_nodeg(*hs, idxg, zrows)
        A = Wsn[:D]
        if last:
            (out,) = _fuse_last(*hs, *ss, deg, A, Bm, Cm)
        else:
            hs = _fuse_mid(*hs, *ss, deg, A, Bm, Cm)
    return out


# speedup vs baseline: 6.8381x; 1.0732x over previous
import functools

import jax
import jax.numpy as jnp
from jax import lax
from jax.experimental import pallas as pl
from jax.experimental.pallas import tpu as pltpu
from jax.experimental.pallas import tpu_sc as plsc

N, E, D = 10000, 160000, 256
DC = 64
NCH = D // DC
NC, NS = 2, 16
ET = E // NS
B = 80
NB = ET // B
NP = 10240
DW = 16
ROWS_T = NP // NS
BN = 1000
GRID_N = N // BN


def _init_body(x_ref, w_ref, *o_refs):
    h = jax.nn.relu(jnp.dot(x_ref[...], w_ref[...],
                            preferred_element_type=jnp.float32))
    o_refs[0][...] = h[:, :D // 2]
    o_refs[1][...] = h[:, D // 2:]


_init_mm = pl.pallas_call(
    _init_body,
    grid=(GRID_N,),
    in_specs=[pl.BlockSpec((BN, D), lambda i: (i, 0)),
              pl.BlockSpec((D, D), lambda i: (0, 0))],
    out_specs=[pl.BlockSpec((BN, D // 2), lambda i: (i, 0))] * 2,
    out_shape=[jax.ShapeDtypeStruct((N, D // 2), jnp.float32)] * 2,
)


def _wcomb_body(wf_ref, wb_ref, wm_ref, wt_ref, bm_ref, cm_ref):
    bm_ref[...] = jnp.dot(wf_ref[...], wm_ref[...],
                          preferred_element_type=jnp.float32)
    cm_ref[...] = jnp.dot(wb_ref[...], wt_ref[...],
                          preferred_element_type=jnp.float32)


_wcomb = pl.pallas_call(
    _wcomb_body,
    out_shape=[jax.ShapeDtypeStruct((D, D), jnp.float32),
               jax.ShapeDtypeStruct((D, D), jnp.float32)],
)


def _fuse_body(*refs):
    h_refs = refs[:2]
    s_refs = refs[2:2 + NCH]
    deg_ref, a_ref, b_ref, c_ref = refs[2 + NCH:6 + NCH]
    out_refs = refs[6 + NCH:]
    h_full = jnp.concatenate([r[...] for r in h_refs], axis=1)
    degf = jnp.maximum(deg_ref[0, :, 0:1], 1.0)
    degb = jnp.maximum(deg_ref[1, :, 0:1], 1.0)
    mf = jnp.concatenate([r[0] for r in s_refs], axis=1) / degf
    mb = jnp.concatenate([r[1] for r in s_refs], axis=1) / degb
    o = jnp.dot(h_full, a_ref[...], preferred_element_type=jnp.float32)
    o += jnp.dot(mf, b_ref[...], preferred_element_type=jnp.float32)
    o += jnp.dot(mb, c_ref[...], preferred_element_type=jnp.float32)
    o = jax.nn.relu(o)
    if len(out_refs) == 2:
        out_refs[0][...] = o[:, :D // 2]
        out_refs[1][...] = o[:, D // 2:]
    else:
        out_refs[0][...] = o


_fuse_in_specs = [
    pl.BlockSpec((BN, D // 2), lambda i: (i, 0)) for _ in range(2)
] + [
    pl.BlockSpec((NC, BN, DC), lambda i: (0, i, 0)) for _ in range(NCH)
] + [
    pl.BlockSpec((NC, BN, DW), lambda i: (0, i, 0)),
    pl.BlockSpec((D, D), lambda i: (0, 0)),
    pl.BlockSpec((D, D), lambda i: (0, 0)),
    pl.BlockSpec((D, D), lambda i: (0, 0)),
]

_fuse_mid = pl.pallas_call(
    _fuse_body,
    grid=(GRID_N,),
    in_specs=_fuse_in_specs,
    out_specs=[pl.BlockSpec((BN, D // 2), lambda i: (i, 0))] * 2,
    out_shape=[jax.ShapeDtypeStruct((N, D // 2), jnp.float32)] * 2,
)

_fuse_last = pl.pallas_call(
    _fuse_body,
    grid=(GRID_N,),
    in_specs=_fuse_in_specs,
    out_specs=[pl.BlockSpec((BN, D), lambda i: (i, 0))],
    out_shape=[jax.ShapeDtypeStruct((N, D), jnp.float32)],
)


NBUF = 6
PREF = 4


def _make_agg_body(want_deg):
    def _agg_body(*refs):
        h_halves = refs[:2]
        if want_deg:
            idxf, zrows, zdeg, ones_h = refs[2:6]
            sum_chunks = refs[6:NCH + 6]
            degout = refs[NCH + 6]
            rest = refs[NCH + 7:]
        else:
            idxf, zrows = refs[2:4]
            sum_chunks = refs[4:NCH + 4]
            rest = refs[NCH + 4:]
        idxe_v, idxo_v, idxs_f, idxs_v = rest[:4]
        rows = rest[4:4 + NBUF]
        if want_deg:
            ones_v, acc, dacc = rest[4 + NBUF:7 + NBUF]
            gsem = rest[7 + NBUF:7 + 2 * NBUF]
            ssem = rest[7 + 2 * NBUF:7 + 3 * NBUF]
        else:
            acc = rest[4 + NBUF]
            gsem = rest[5 + NBUF:5 + 2 * NBUF]
            ssem = rest[5 + 2 * NBUF:5 + 3 * NBUF]
        c = lax.axis_index("c")
        s = lax.axis_index("s")
        row0 = s * ROWS_T

        pltpu.sync_copy(idxf.at[c, s], idxe_v)
        pltpu.sync_copy(idxf.at[1 - c, s], idxs_f)
        if want_deg:
            pltpu.sync_copy(ones_h, ones_v)

        def tloop(i, carry):
            v = idxe_v[pl.ds(i * 16, 16)]
            idxo_v[pl.ds(i * 16, 16)] = v * 2 + 1
            idxe_v[pl.ds(i * 16, 16)] = v * 2
            r = lax.div(i, B // 16)
            cc = lax.rem(i, B // 16) * 16
            idxs_v[r, pl.ds(cc, 16)] = idxs_f[pl.ds(i * 16, 16)]
            return carry

        lax.fori_loop(0, ET // 16, tloop, 0)

        for chunk in range(NCH):
            h_ref = h_halves[chunk // 2]
            gidx_v = (idxe_v, idxo_v)[chunk % 2]
            out_ref = sum_chunks[chunk]

            pltpu.sync_copy(zrows.at[pl.ds(row0, ROWS_T)],
                            acc.at[pl.ds(row0, ROWS_T)])
            if want_deg and chunk == 0:
                pltpu.sync_copy(zdeg.at[pl.ds(row0, ROWS_T)],
                                dacc.at[pl.ds(row0, ROWS_T)])
            plsc.subcore_barrier()

            for j in range(PREF):
                pltpu.async_copy(h_ref.at[gidx_v.at[pl.ds(j * B, B)]],
                                 rows[j], gsem[j])

            def loop_body(b, carry, chunk=chunk, h_ref=h_ref,
                          gidx_v=gidx_v):
                nxt = b + PREF

                @pl.when(nxt < NB)
                def _():
                    for j in range(NBUF):
                        @pl.when(lax.rem(nxt, NBUF) == j)
                        def _(j=j):
                            @pl.when(nxt >= NBUF)
                            def _():
                                pltpu.make_async_copy(
                                    rows[j], acc.at[idxs_v.at[b]],
                                    ssem[j]).wait()
                            pltpu.async_copy(
                                h_ref.at[gidx_v.at[pl.ds(nxt * B, B)]],
                                rows[j], gsem[j])

                for j in range(NBUF):
                    @pl.when(lax.rem(b, NBUF) == j)
                    def _(j=j):
                        pltpu.make_async_copy(
                            h_ref.at[gidx_v.at[pl.ds(b * B, B)]],
                            rows[j], gsem[j]).wait()
                        pltpu.async_copy(rows[j], acc.at[idxs_v.at[b]],
                                         ssem[j], add=True)

                if want_deg and chunk == 0:
                    pltpu.sync_copy(ones_v, dacc.at[idxs_v.at[b]], add=True)
                return carry

            lax.fori_loop(0, NB, loop_body, 0)

            for j in range(NBUF):
                pltpu.make_async_copy(rows[j], acc.at[idxs_v.at[0]],
                                      ssem[j]).wait()

            plsc.subcore_barrier()
            pltpu.sync_copy(acc.at[pl.ds(row0, ROWS_T)],
                            out_ref.at[c, pl.ds(row0, ROWS_T)])
            if want_deg and chunk == 0:
                pltpu.sync_copy(dacc.at[pl.ds(row0, ROWS_T)],
                                degout.at[c, pl.ds(row0, ROWS_T)])
            plsc.subcore_barrier()

    return _agg_body


_sc_mesh = plsc.VectorSubcoreMesh(core_axis_name="c", subcore_axis_name="s",
                                  num_cores=NC, num_subcores=NS)

_agg_deg = pl.kernel(
    _make_agg_body(True),
    out_type=[jax.ShapeDtypeStruct((NC, NP, DC), jnp.float32)] * NCH +
             [jax.ShapeDtypeStruct((NC, NP, DW), jnp.float32)],
    mesh=_sc_mesh,
    scratch_types=[pltpu.VMEM((ET,), jnp.int32),
                   pltpu.VMEM((ET,), jnp.int32),
                   pltpu.VMEM((ET,), jnp.int32),
                   pltpu.VMEM((NB, B), jnp.int32)] +
                  [pltpu.VMEM((B, DC), jnp.float32)] * NBUF +
                  [pltpu.VMEM((B, DW), jnp.float32),
                   pltpu.VMEM_SHARED((NP, DC), jnp.float32),
                   pltpu.VMEM_SHARED((NP, DW), jnp.float32)] +
                  [pltpu.SemaphoreType.DMA] * (2 * NBUF),
    compiler_params=pltpu.CompilerParams(use_tc_tiling_on_sc=False),
)

_agg_nodeg = pl.kernel(
    _make_agg_body(False),
    out_type=[jax.ShapeDtypeStruct((NC, NP, DC), jnp.float32)] * NCH,
    mesh=_sc_mesh,
    scratch_types=[pltpu.VMEM((ET,), jnp.int32),
                   pltpu.VMEM((ET,), jnp.int32),
                   pltpu.VMEM((ET,), jnp.int32),
                   pltpu.VMEM((NB, B), jnp.int32)] +
                  [pltpu.VMEM((B, DC), jnp.float32)] * NBUF +
                  [pltpu.VMEM_SHARED((NP, DC), jnp.float32)] +
                  [pltpu.SemaphoreType.DMA] * (2 * NBUF),
    compiler_params=pltpu.CompilerParams(use_tc_tiling_on_sc=False),
)


def kernel(x, edge_index, W_init, W_f0, W_b0, W_sn0, W_f1, W_b1, W_sn1):
    idxf = edge_index.reshape(NC, NS, ET)
    zrows = jnp.zeros((NP, DC), jnp.float32)
    zdeg = jnp.zeros((NP, DW), jnp.float32)
    ones_h = jnp.ones((B, DW), jnp.float32)

    hs = _init_mm(x, W_init)
    out = None
    deg = None
    for (Wf, Wb, Wsn, last) in ((W_f0, W_b0, W_sn0, False),
                                (W_f1, W_b1, W_sn1, True)):
        Bm, Cm = _wcomb(Wf, Wb, Wsn[D:2 * D], Wsn[2 * D:])
        hs_sc = [jnp.reshape(h, (2 * N, D // 4)) for h in hs]
        if deg is None:
            aggout = _agg_deg(*hs_sc, idxf, zrows, zdeg, ones_h)
            ss, deg = aggout[:NCH], aggout[NCH]
        else:
            ss = _agg_nodeg(*hs_sc, idxf, zrows)
        A = Wsn[:D]
        if last:
            (out,) = _fuse_last(*hs, *ss, deg, A, Bm, Cm)
        else:
            hs = _fuse_mid(*hs, *ss, deg, A, Bm, Cm)
    return out


# speedup vs baseline: 6.8441x; 1.0009x over previous
import functools

import jax
import jax.numpy as jnp
from jax import lax
from jax.experimental import pallas as pl
from jax.experimental.pallas import tpu as pltpu
from jax.experimental.pallas import tpu_sc as plsc

N, E, D = 10000, 160000, 256
DC = 64
NCH = D // DC
NC, NS = 2, 16
ET = E // NS
B = 80
NB = ET // B
NP = 10240
DW = 16
ROWS_T = NP // NS
BN = 1000
GRID_N = N // BN


def _init_body(x_ref, w_ref, *o_refs):
    h = jax.nn.relu(jnp.dot(x_ref[...], w_ref[...],
                            preferred_element_type=jnp.float32))
    o_refs[0][...] = h[:, :D // 2]
    o_refs[1][...] = h[:, D // 2:]


_init_mm = pl.pallas_call(
    _init_body,
    grid=(GRID_N,),
    in_specs=[pl.BlockSpec((BN, D), lambda i: (i, 0)),
              pl.BlockSpec((D, D), lambda i: (0, 0))],
    out_specs=[pl.BlockSpec((BN, D // 2), lambda i: (i, 0))] * 2,
    out_shape=[jax.ShapeDtypeStruct((N, D // 2), jnp.float32)] * 2,
)


def _wcomb_body(wf_ref, wb_ref, wm_ref, wt_ref, bm_ref, cm_ref):
    bm_ref[...] = jnp.dot(wf_ref[...], wm_ref[...],
                          preferred_element_type=jnp.float32)
    cm_ref[...] = jnp.dot(wb_ref[...], wt_ref[...],
                          preferred_element_type=jnp.float32)


_wcomb = pl.pallas_call(
    _wcomb_body,
    out_shape=[jax.ShapeDtypeStruct((D, D), jnp.float32),
               jax.ShapeDtypeStruct((D, D), jnp.float32)],
)


def _fuse_body(*refs):
    h_refs = refs[:2]
    s_refs = refs[2:2 + NCH]
    deg_ref, a_ref, b_ref, c_ref = refs[2 + NCH:6 + NCH]
    out_refs = refs[6 + NCH:]
    h_full = jnp.concatenate([r[...] for r in h_refs], axis=1)
    degf = jnp.maximum(deg_ref[0, :, 0:1], 1.0)
    degb = jnp.maximum(deg_ref[1, :, 0:1], 1.0)
    mf = jnp.concatenate([r[0] for r in s_refs], axis=1) / degf
    mb = jnp.concatenate([r[1] for r in s_refs], axis=1) / degb
    o = jnp.dot(h_full, a_ref[...], preferred_element_type=jnp.float32)
    o += jnp.dot(mf, b_ref[...], preferred_element_type=jnp.float32)
    o += jnp.dot(mb, c_ref[...], preferred_element_type=jnp.float32)
    o = jax.nn.relu(o)
    if len(out_refs) == 2:
        out_refs[0][...] = o[:, :D // 2]
        out_refs[1][...] = o[:, D // 2:]
    else:
        out_refs[0][...] = o


_fuse_in_specs = [
    pl.BlockSpec((BN, D // 2), lambda i: (i, 0)) for _ in range(2)
] + [
    pl.BlockSpec((NC, BN, DC), lambda i: (0, i, 0)) for _ in range(NCH)
] + [
    pl.BlockSpec((NC, BN, DW), lambda i: (0, i, 0)),
    pl.BlockSpec((D, D), lambda i: (0, 0)),
    pl.BlockSpec((D, D), lambda i: (0, 0)),
    pl.BlockSpec((D, D), lambda i: (0, 0)),
]

_fuse_mid = pl.pallas_call(
    _fuse_body,
    grid=(GRID_N,),
    in_specs=_fuse_in_specs,
    out_specs=[pl.BlockSpec((BN, D // 2), lambda i: (i, 0))] * 2,
    out_shape=[jax.ShapeDtypeStruct((N, D // 2), jnp.float32)] * 2,
)

_fuse_last = pl.pallas_call(
    _fuse_body,
    grid=(GRID_N,),
    in_specs=_fuse_in_specs,
    out_specs=[pl.BlockSpec((BN, D), lambda i: (i, 0))],
    out_shape=[jax.ShapeDtypeStruct((N, D), jnp.float32)],
)


NBUF = 7
PREF = 5


def _make_agg_body(want_deg):
    def _agg_body(*refs):
        h_halves = refs[:2]
        if want_deg:
            idxf, zrows, zdeg, ones_h = refs[2:6]
            sum_chunks = refs[6:NCH + 6]
            degout = refs[NCH + 6]
            rest = refs[NCH + 7:]
        else:
            idxf, zrows = refs[2:4]
            sum_chunks = refs[4:NCH + 4]
            rest = refs[NCH + 4:]
        idxe_v, idxo_v, idxs_f, idxs_v = rest[:4]
        rows = rest[4:4 + NBUF]
        if want_deg:
            ones_v, acc, dacc = rest[4 + NBUF:7 + NBUF]
            gsem = rest[7 + NBUF:7 + 2 * NBUF]
            ssem = rest[7 + 2 * NBUF:7 + 3 * NBUF]
        else:
            acc = rest[4 + NBUF]
            gsem = rest[5 + NBUF:5 + 2 * NBUF]
            ssem = rest[5 + 2 * NBUF:5 + 3 * NBUF]
        c = lax.axis_index("c")
        s = lax.axis_index("s")
        row0 = s * ROWS_T

        pltpu.sync_copy(idxf.at[c, s], idxe_v)
        pltpu.sync_copy(idxf.at[1 - c, s], idxs_f)
        if want_deg:
            pltpu.sync_copy(ones_h, ones_v)

        def tloop(i, carry):
            v = idxe_v[pl.ds(i * 16, 16)]
            idxo_v[pl.ds(i * 16, 16)] = v * 2 + 1
            idxe_v[pl.ds(i * 16, 16)] = v * 2
            r = lax.div(i, B // 16)
            cc = lax.rem(i, B // 16) * 16
            idxs_v[r, pl.ds(cc, 16)] = idxs_f[pl.ds(i * 16, 16)]
            return carry

        lax.fori_loop(0, ET // 16, tloop, 0)

        for chunk in range(NCH):
            h_ref = h_halves[chunk // 2]
            gidx_v = (idxe_v, idxo_v)[chunk % 2]
            out_ref = sum_chunks[chunk]

            pltpu.sync_copy(zrows.at[pl.ds(row0, ROWS_T)],
                            acc.at[pl.ds(row0, ROWS_T)])
            if want_deg and chunk == 0:
                pltpu.sync_copy(zdeg.at[pl.ds(row0, ROWS_T)],
                                dacc.at[pl.ds(row0, ROWS_T)])
            plsc.subcore_barrier()

            for j in range(PREF):
                pltpu.async_copy(h_ref.at[gidx_v.at[pl.ds(j * B, B)]],
                                 rows[j], gsem[j])

            def loop_body(b, carry, chunk=chunk, h_ref=h_ref,
                          gidx_v=gidx_v):
                nxt = b + PREF

                @pl.when(nxt < NB)
                def _():
                    for j in range(NBUF):
                        @pl.when(lax.rem(nxt, NBUF) == j)
                        def _(j=j):
                            @pl.when(nxt >= NBUF)
                            def _():
                                pltpu.make_async_copy(
                                    rows[j], acc.at[idxs_v.at[b]],
                                    ssem[j]).wait()
                            pltpu.async_copy(
                                h_ref.at[gidx_v.at[pl.ds(nxt * B, B)]],
                                rows[j], gsem[j])

                for j in range(NBUF):
                    @pl.when(lax.rem(b, NBUF) == j)
                    def _(j=j):
                        pltpu.make_async_copy(
                            h_ref.at[gidx_v.at[pl.ds(b * B, B)]],
                            rows[j], gsem[j]).wait()
                        pltpu.async_copy(rows[j], acc.at[idxs_v.at[b]],
                                         ssem[j], add=True)

                if want_deg and chunk == 0:
                    pltpu.sync_copy(ones_v, dacc.at[idxs_v.at[b]], add=True)
                return carry

            lax.fori_loop(0, NB, loop_body, 0)

            for j in range(NBUF):
                pltpu.make_async_copy(rows[j], acc.at[idxs_v.at[0]],
                                      ssem[j]).wait()

            plsc.subcore_barrier()
            pltpu.sync_copy(acc.at[pl.ds(row0, ROWS_T)],
                            out_ref.at[c, pl.ds(row0, ROWS_T)])
            if want_deg and chunk == 0:
                pltpu.sync_copy(dacc.at[pl.ds(row0, ROWS_T)],
                                degout.at[c, pl.ds(row0, ROWS_T)])
            plsc.subcore_barrier()

    return _agg_body


_sc_mesh = plsc.VectorSubcoreMesh(core_axis_name="c", subcore_axis_name="s",
                                  num_cores=NC, num_subcores=NS)

_agg_deg = pl.kernel(
    _make_agg_body(True),
    out_type=[jax.ShapeDtypeStruct((NC, NP, DC), jnp.float32)] * NCH +
             [jax.ShapeDtypeStruct((NC, NP, DW), jnp.float32)],
    mesh=_sc_mesh,
    scratch_types=[pltpu.VMEM((ET,), jnp.int32),
                   pltpu.VMEM((ET,), jnp.int32),
                   pltpu.VMEM((ET,), jnp.int32),
                   pltpu.VMEM((NB, B), jnp.int32)] +
                  [pltpu.VMEM((B, DC), jnp.float32)] * NBUF +
                  [pltpu.VMEM((B, DW), jnp.float32),
                   pltpu.VMEM_SHARED((NP, DC), jnp.float32),
                   pltpu.VMEM_SHARED((NP, DW), jnp.float32)] +
                  [pltpu.SemaphoreType.DMA] * (2 * NBUF),
    compiler_params=pltpu.CompilerParams(use_tc_tiling_on_sc=False),
)

_agg_nodeg = pl.kernel(
    _make_agg_body(False),
    out_type=[jax.ShapeDtypeStruct((NC, NP, DC), jnp.float32)] * NCH,
    mesh=_sc_mesh,
    scratch_types=[pltpu.VMEM((ET,), jnp.int32),
                   pltpu.VMEM((ET,), jnp.int32),
                   pltpu.VMEM((ET,), jnp.int32),
                   pltpu.VMEM((NB, B), jnp.int32)] +
                  [pltpu.VMEM((B, DC), jnp.float32)] * NBUF +
                  [pltpu.VMEM_SHARED((NP, DC), jnp.float32)] +
                  [pltpu.SemaphoreType.DMA] * (2 * NBUF),
    compiler_params=pltpu.CompilerParams(use_tc_tiling_on_sc=False),
)


def kernel(x, edge_index, W_init, W_f0, W_b0, W_sn0, W_f1, W_b1, W_sn1):
    idxf = edge_index.reshape(NC, NS, ET)
    zrows = jnp.zeros((NP, DC), jnp.float32)
    zdeg = jnp.zeros((NP, DW), jnp.float32)
    ones_h = jnp.ones((B, DW), jnp.float32)

    hs = _init_mm(x, W_init)
    out = None
    deg = None
    for (Wf, Wb, Wsn, last) in ((W_f0, W_b0, W_sn0, False),
                                (W_f1, W_b1, W_sn1, True)):
        Bm, Cm = _wcomb(Wf, Wb, Wsn[D:2 * D], Wsn[2 * D:])
        hs_sc = [jnp.reshape(h, (2 * N, D // 4)) for h in hs]
        if deg is None:
            aggout = _agg_deg(*hs_sc, idxf, zrows, zdeg, ones_h)
            ss, deg = aggout[:NCH], aggout[NCH]
        else:
            ss = _agg_nodeg(*hs_sc, idxf, zrows)
        A = Wsn[:D]
        if last:
            (out,) = _fuse_last(*hs, *ss, deg, A, Bm, Cm)
        else:
            hs = _fuse_mid(*hs, *ss, deg, A, Bm, Cm)
    return out
